# baseline (speedup 1.0000x reference)
import math

import jax
import jax.numpy as jnp
from jax.experimental import pallas as pl

N_Q = 8
BINS = 1024
DIM = 256
CODE_DIM = 8


def _rvq_kernel(x_ref, pw_ref, pb_ref, pow_ref, pob_ref, cb_ref, c2h_ref,
                q_out_ref, codes_ref, loss_ref):
    Tt = x_ref.shape[2]
    H = Tt // 2
    row_iota = jax.lax.broadcasted_iota(jnp.int32, (BINS, H), 0)

    def stage(i, r):
        P = pw_ref[i]
        xp = jax.lax.dot_general(P, r, (((1,), (0,)), ((), ())),
                                 preferred_element_type=jnp.float32)
        xp = xp + pb_ref[i][:, None]
        s = jax.lax.dot_general(cb_ref[i], xp, (((1,), (0,)), ((), ())),
                                 preferred_element_type=jnp.float32)
        sc = c2h_ref[i][:, None] - s
        m = jnp.min(sc, axis=0)
        masked = jnp.where(sc == m[None, :], row_iota, BINS)
        idx = jnp.min(masked, axis=0)
        onehot = (row_iota == idx[None, :]).astype(jnp.float32)
        q = jax.lax.dot_general(cb_ref[i], onehot, (((0,), (0,)), ((), ())),
                                preferred_element_type=jnp.float32)
        lp = jnp.sum((q - xp) ** 2, axis=0)
        qo = jax.lax.dot_general(pow_ref[i], q, (((1,), (0,)), ((), ())),
                                 preferred_element_type=jnp.float32)
        qo = qo + pob_ref[i][:, None]
        return r - qo, qo, idx, lp

    halves = []
    for h in range(2):
        r = x_ref[0, :, h * H:(h + 1) * H]
        halves.append({"r": r, "qacc": jnp.zeros_like(r), "idx": [], "lp": []})

    for i in range(N_Q):
        for st in halves:
            r, qo, idx, lp = stage(i, st["r"])
            st["r"] = r
            st["qacc"] = st["qacc"] + qo
            st["idx"].append(idx)
            st["lp"].append(lp)

    for h, st in enumerate(halves):
        sl = pl.ds(h * H, H)
        q_out_ref[0, :, sl] = st["qacc"]
        codes_ref[0, :, sl] = jnp.stack(st["idx"], axis=0)
        loss_ref[0, :, sl] = jnp.stack(st["lp"], axis=0)


def kernel(x, frame_rate, proj_in_w, proj_in_b, proj_out_w, proj_out_b, codebooks):
    B, D, T = x.shape
    Tt = 1024
    grid = (B, T // Tt)

    c2h = 0.5 * jnp.sum(codebooks * codebooks, axis=-1)

    quantized, codes_tmp, loss_parts = pl.pallas_call(
        _rvq_kernel,
        grid=grid,
        in_specs=[
            pl.BlockSpec((1, D, Tt), lambda b, t: (b, 0, t)),
            pl.BlockSpec((N_Q, CODE_DIM, D), lambda b, t: (0, 0, 0)),
            pl.BlockSpec((N_Q, CODE_DIM), lambda b, t: (0, 0)),
            pl.BlockSpec((N_Q, D, CODE_DIM), lambda b, t: (0, 0, 0)),
            pl.BlockSpec((N_Q, D), lambda b, t: (0, 0)),
            pl.BlockSpec((N_Q, BINS, CODE_DIM), lambda b, t: (0, 0, 0)),
            pl.BlockSpec((N_Q, BINS), lambda b, t: (0, 0)),
        ],
        out_specs=[
            pl.BlockSpec((1, D, Tt), lambda b, t: (b, 0, t)),
            pl.BlockSpec((1, N_Q, Tt), lambda b, t: (b, 0, t)),
            pl.BlockSpec((1, N_Q, Tt), lambda b, t: (b, 0, t)),
        ],
        out_shape=[
            jax.ShapeDtypeStruct((B, D, T), jnp.float32),
            jax.ShapeDtypeStruct((B, N_Q, T), jnp.int32),
            jax.ShapeDtypeStruct((B, N_Q, T), jnp.float32),
        ],
    )(x, proj_in_w, proj_in_b, proj_out_w, proj_out_b, codebooks, c2h)

    codes = jnp.transpose(codes_tmp, (1, 0, 2))
    commit_loss = jnp.sum(loss_parts, axis=(0, 2)) / (B * T * CODE_DIM)
    bw = jnp.asarray(N_Q * math.log2(BINS) * frame_rate, x.dtype)
    return quantized, codes, bw, commit_loss


# speedup vs baseline: 1.3117x; 1.3117x over previous
import math

import jax
import jax.numpy as jnp
from jax.experimental import pallas as pl

N_Q = 8
BINS = 1024
DIM = 256
CODE_DIM = 8


def _rvq_kernel(x_ref, pw_ref, pb_ref, pow_ref, pob_ref, cb_ref, c2h_ref,
                q_out_ref, codes_ref, loss_ref):
    Tt = x_ref.shape[2]
    NC = 4
    H = Tt // NC
    row_iota = jax.lax.broadcasted_iota(jnp.int32, (BINS, H), 0)

    def stage(i, r):
        P = pw_ref[i]
        xp = jax.lax.dot_general(P, r, (((1,), (0,)), ((), ())),
                                 preferred_element_type=jnp.float32)
        xp = xp + pb_ref[i][:, None]
        s = jax.lax.dot_general(cb_ref[i], xp, (((1,), (0,)), ((), ())),
                                 preferred_element_type=jnp.float32)
        sc = c2h_ref[i][:, None] - s
        idx = jnp.argmin(sc, axis=0)
        onehot = (row_iota == idx[None, :]).astype(jnp.float32)
        q = jax.lax.dot_general(cb_ref[i], onehot, (((0,), (0,)), ((), ())),
                                preferred_element_type=jnp.float32)
        lp = jnp.sum((q - xp) ** 2, axis=0)
        qo = jax.lax.dot_general(pow_ref[i], q, (((1,), (0,)), ((), ())),
                                 preferred_element_type=jnp.float32)
        qo = qo + pob_ref[i][:, None]
        return r - qo, qo, idx, lp

    halves = []
    for h in range(NC):
        r = x_ref[0, :, h * H:(h + 1) * H]
        halves.append({"r": r, "qacc": jnp.zeros_like(r), "idx": [], "lp": []})

    for i in range(N_Q):
        for st in halves:
            r, qo, idx, lp = stage(i, st["r"])
            st["r"] = r
            st["qacc"] = st["qacc"] + qo
            st["idx"].append(idx)
            st["lp"].append(lp)

    for h, st in enumerate(halves):
        sl = pl.ds(h * H, H)
        q_out_ref[0, :, sl] = st["qacc"]
        codes_ref[0, :, sl] = jnp.stack(st["idx"], axis=0)
        loss_ref[0, :, sl] = jnp.stack(st["lp"], axis=0)


def kernel(x, frame_rate, proj_in_w, proj_in_b, proj_out_w, proj_out_b, codebooks):
    B, D, T = x.shape
    Tt = 2048
    grid = (B, T // Tt)

    c2h = 0.5 * jnp.sum(codebooks * codebooks, axis=-1)

    quantized, codes_tmp, loss_parts = pl.pallas_call(
        _rvq_kernel,
        grid=grid,
        in_specs=[
            pl.BlockSpec((1, D, Tt), lambda b, t: (b, 0, t)),
            pl.BlockSpec((N_Q, CODE_DIM, D), lambda b, t: (0, 0, 0)),
            pl.BlockSpec((N_Q, CODE_DIM), lambda b, t: (0, 0)),
            pl.BlockSpec((N_Q, D, CODE_DIM), lambda b, t: (0, 0, 0)),
            pl.BlockSpec((N_Q, D), lambda b, t: (0, 0)),
            pl.BlockSpec((N_Q, BINS, CODE_DIM), lambda b, t: (0, 0, 0)),
            pl.BlockSpec((N_Q, BINS), lambda b, t: (0, 0)),
        ],
        out_specs=[
            pl.BlockSpec((1, D, Tt), lambda b, t: (b, 0, t)),
            pl.BlockSpec((1, N_Q, Tt), lambda b, t: (b, 0, t)),
            pl.BlockSpec((1, N_Q, Tt), lambda b, t: (b, 0, t)),
        ],
        out_shape=[
            jax.ShapeDtypeStruct((B, D, T), jnp.float32),
            jax.ShapeDtypeStruct((B, N_Q, T), jnp.int32),
            jax.ShapeDtypeStruct((B, N_Q, T), jnp.float32),
        ],
    )(x, proj_in_w, proj_in_b, proj_out_w, proj_out_b, codebooks, c2h)

    codes = jnp.transpose(codes_tmp, (1, 0, 2))
    commit_loss = jnp.sum(loss_parts, axis=(0, 2)) / (B * T * CODE_DIM)
    bw = jnp.asarray(N_Q * math.log2(BINS) * frame_rate, x.dtype)
    return quantized, codes, bw, commit_loss


# speedup vs baseline: 1.8629x; 1.4201x over previous
import math

import jax
import jax.numpy as jnp
from jax.experimental import pallas as pl

N_Q = 8
BINS = 1024
DIM = 256
CODE_DIM = 8
NQC = N_Q * CODE_DIM


def _rvq_kernel(x_ref, ps_ref, bia_ref, wall_ref, bos_ref, cb_ref, c2h_ref,
                ga_ref, q_out_ref, codes_ref, loss_ref):
    Tt = x_ref.shape[2]
    NC = 4
    H = Tt // NC
    row_iota = jax.lax.broadcasted_iota(jnp.int32, (BINS, H), 0)

    def stage(i, xpf):
        xp = xpf[CODE_DIM * i:CODE_DIM * (i + 1)] + bia_ref[i][:, None]
        s = jax.lax.dot_general(cb_ref[i], xp, (((1,), (0,)), ((), ())),
                                preferred_element_type=jnp.float32)
        sc = c2h_ref[i][:, None] - s
        idx = jnp.argmin(sc, axis=0)
        onehot = (row_iota == idx[None, :]).astype(jnp.float32)
        fused = jax.lax.dot_general(ga_ref[i], onehot, (((1,), (0,)), ((), ())),
                                    preferred_element_type=jnp.float32)
        q = fused[:CODE_DIM]
        lp = jnp.sum((q - xp) ** 2, axis=0)
        return xpf - fused[CODE_DIM:], q, idx, lp

    chains = []
    for h in range(NC):
        r0 = x_ref[0, :, h * H:(h + 1) * H]
        xpf = jax.lax.dot_general(ps_ref[...], r0, (((1,), (0,)), ((), ())),
                                  preferred_element_type=jnp.float32)
        chains.append({"xpf": xpf, "q": [], "idx": [], "lp": []})

    for i in range(N_Q):
        for st in chains:
            xpf, q, idx, lp = stage(i, st["xpf"])
            st["xpf"] = xpf
            st["q"].append(q)
            st["idx"].append(idx)
            st["lp"].append(lp)

    for h, st in enumerate(chains):
        sl = pl.ds(h * H, H)
        qall = jnp.concatenate(st["q"], axis=0)
        qo = jax.lax.dot_general(wall_ref[...], qall, (((1,), (0,)), ((), ())),
                                 preferred_element_type=jnp.float32)
        q_out_ref[0, :, sl] = qo + bos_ref[...]
        codes_ref[0, :, sl] = jnp.stack(st["idx"], axis=0)
        loss_ref[0, :, sl] = jnp.stack(st["lp"], axis=0)


def kernel(x, frame_rate, proj_in_w, proj_in_b, proj_out_w, proj_out_b, codebooks):
    B, D, T = x.shape
    Tt = 2048
    grid = (B, T // Tt)
    f32 = jnp.float32

    c2h = 0.5 * jnp.sum(codebooks * codebooks, axis=-1)
    pstack = proj_in_w.reshape(NQC, D)
    cstack = jnp.einsum("kd,ndc->nkc", pstack, proj_out_w)
    g = jnp.einsum("nkc,nbc->nkb", cstack, codebooks)
    gaug = jnp.concatenate([jnp.transpose(codebooks, (0, 2, 1)), g], axis=1)
    cumbo = jnp.cumsum(proj_out_b, axis=0) - proj_out_b
    bia = proj_in_b - jnp.einsum("jcd,jd->jc", proj_in_w, cumbo)
    wall = jnp.transpose(proj_out_w, (1, 0, 2)).reshape(D, NQC)
    bosum = jnp.sum(proj_out_b, axis=0)[:, None]

    quantized, codes_tmp, loss_parts = pl.pallas_call(
        _rvq_kernel,
        grid=grid,
        in_specs=[
            pl.BlockSpec((1, D, Tt), lambda b, t: (b, 0, t)),
            pl.BlockSpec((NQC, D), lambda b, t: (0, 0)),
            pl.BlockSpec((N_Q, CODE_DIM), lambda b, t: (0, 0)),
            pl.BlockSpec((D, NQC), lambda b, t: (0, 0)),
            pl.BlockSpec((D, 1), lambda b, t: (0, 0)),
            pl.BlockSpec((N_Q, BINS, CODE_DIM), lambda b, t: (0, 0, 0)),
            pl.BlockSpec((N_Q, BINS), lambda b, t: (0, 0)),
            pl.BlockSpec((N_Q, CODE_DIM + NQC, BINS), lambda b, t: (0, 0, 0)),
        ],
        out_specs=[
            pl.BlockSpec((1, D, Tt), lambda b, t: (b, 0, t)),
            pl.BlockSpec((1, N_Q, Tt), lambda b, t: (b, 0, t)),
            pl.BlockSpec((1, N_Q, Tt), lambda b, t: (b, 0, t)),
        ],
        out_shape=[
            jax.ShapeDtypeStruct((B, D, T), f32),
            jax.ShapeDtypeStruct((B, N_Q, T), jnp.int32),
            jax.ShapeDtypeStruct((B, N_Q, T), f32),
        ],
    )(x, pstack, bia, wall, bosum, codebooks, c2h, gaug)

    codes = jnp.transpose(codes_tmp, (1, 0, 2))
    commit_loss = jnp.sum(loss_parts, axis=(0, 2)) / (B * T * CODE_DIM)
    bw = jnp.asarray(N_Q * math.log2(BINS) * frame_rate, x.dtype)
    return quantized, codes, bw, commit_loss
